# baseline (speedup 1.0000x reference)
import numpy as np

import jax
import jax.numpy as jnp
from jax.experimental import pallas as pl
from jax.experimental.pallas import tpu as pltpu


def _masks(nu, ng, parity):
    u = np.arange(nu)[:, None]
    g = np.arange(ng)[None, :]
    dxm = u + 1 - (2 * g + parity)
    return [(dxm == dx).astype(np.float32) for dx in range(3)]


_M1_MASKS = {p: _masks(28, 14, p) for p in (0, 1)}
_M2_MASKS = {p: _masks(14, 7, p) for p in (0, 1)}


def _band_conv1(w1, parity):
    w1t = w1.reshape(32, 3, 3).transpose(1, 2, 0)
    m = 0.0
    for dx in range(3):
        m = m + (_M1_MASKS[parity][dx][None, :, :, None]
                 * w1t[:, dx][:, None, None, :])
    return m.reshape(84, 448)


def _band_conv2(w2, parity):
    w2t = w2.transpose(2, 3, 1, 0)
    m = 0.0
    for dx in range(3):
        m = m + (_M2_MASKS[parity][dx][None, :, None, :, None]
                 * w2t[:, dx][:, None, :, None, :])
    return m.reshape(1344, 448)


def _make_net_kernel(bt):
    bf16 = jnp.bfloat16
    f32 = jnp.float32

    hb = bt // 8

    def _half(xpt, m1e_ref, m1o_ref, m2e_ref, m2o_ref,
              b1_ref, b2_ref, wf2_ref, bfc_ref):
        xc1 = jnp.concatenate([xpt[0:28], xpt[1:29], xpt[2:30]],
                              axis=-1).reshape(28 * hb, 84)
        re = jnp.dot(xc1, m1e_ref[...], preferred_element_type=f32)
        ro = jnp.dot(xc1, m1o_ref[...], preferred_element_type=f32)
        r = jnp.maximum(jnp.maximum(re, ro) + b1_ref[...], 0.0)
        r = r.reshape(14, 2, hb, 448)
        a1 = jnp.maximum(r[:, 0], r[:, 1]).astype(bf16)
        zrow = jnp.zeros((1, hb, 448), bf16)
        z = jnp.concatenate([zrow, a1, zrow], axis=0)
        xc2 = jnp.concatenate([z[0:14], z[1:15], z[2:16]],
                              axis=-1).reshape(14 * hb, 1344)
        se = jnp.dot(xc2, m2e_ref[...], preferred_element_type=f32)
        so = jnp.dot(xc2, m2o_ref[...], preferred_element_type=f32)
        s = jnp.maximum(jnp.maximum(se, so) + b2_ref[...], 0.0)
        s = s.reshape(7, 2, hb, 448)
        p = jnp.maximum(s[:, 0], s[:, 1]).astype(bf16)
        acc = jnp.dot(p[0], wf2_ref[0], preferred_element_type=f32)
        for h in range(1, 7):
            acc = acc + jnp.dot(p[h], wf2_ref[h], preferred_element_type=f32)
        zl = acc + bfc_ref[...]
        m = jnp.max(zl, axis=-1, keepdims=True)
        lse = m + jnp.log(jnp.sum(jnp.exp(zl - m), axis=-1, keepdims=True))
        return zl - lse

    def _net_kernel(xpt_ref, m1e_ref, m1o_ref, m2e_ref, m2o_ref,
                    b1_ref, b2_ref, wf2_ref, bfc_ref, o_ref):
        xpt = xpt_ref[0]
        args = (m1e_ref, m1o_ref, m2e_ref, m2o_ref,
                b1_ref, b2_ref, wf2_ref, bfc_ref)
        outs = [_half(xpt[:, c * hb:(c + 1) * hb], *args) for c in range(8)]
        o_ref[...] = jnp.concatenate(outs, axis=0)

    return _net_kernel


def kernel(x, w1, b1, w2, b2, wf, bf):
    if x.ndim != 4:
        raise ValueError("Expected input to a 4D tensor")
    if x.shape[1] != 1 or x.shape[2] != 28 or x.shape[3] != 28:
        raise ValueError("Expected each sample to have shape [1, 28, 28]")
    B = x.shape[0]
    bf16 = jnp.bfloat16
    f32 = jnp.float32
    bt = next(t for t in (256, 128, 64, 32, 16, 8, 4, 2, 1) if B % t == 0)
    nt = B // bt

    xpt = jnp.pad(x.reshape(B, 28, 28), ((0, 0), (1, 1), (0, 0)))
    xpt = xpt.reshape(nt, bt, 30, 28).transpose(0, 2, 1, 3).astype(bf16)
    m1e = _band_conv1(w1, 0).astype(bf16)
    m1o = _band_conv1(w1, 1).astype(bf16)
    m2e = _band_conv2(w2, 0).astype(bf16)
    m2o = _band_conv2(w2, 1).astype(bf16)
    b1row = jnp.tile(b1, 14).reshape(1, 448)
    b2row = jnp.tile(b2, 7).reshape(1, 448)
    wf2 = wf.reshape(10, 64, 7, 7).transpose(2, 3, 1, 0).reshape(7, 448, 10)
    wf2 = wf2.astype(bf16)
    bfc = bf.reshape(1, 10)

    return pl.pallas_call(
        _make_net_kernel(bt),
        out_shape=jax.ShapeDtypeStruct((B, 10), f32),
        grid_spec=pltpu.PrefetchScalarGridSpec(
            num_scalar_prefetch=0,
            grid=(nt,),
            in_specs=[
                pl.BlockSpec((1, 30, bt, 28), lambda i: (i, 0, 0, 0)),
                pl.BlockSpec((84, 448), lambda i: (0, 0)),
                pl.BlockSpec((84, 448), lambda i: (0, 0)),
                pl.BlockSpec((1344, 448), lambda i: (0, 0)),
                pl.BlockSpec((1344, 448), lambda i: (0, 0)),
                pl.BlockSpec((1, 448), lambda i: (0, 0)),
                pl.BlockSpec((1, 448), lambda i: (0, 0)),
                pl.BlockSpec((7, 448, 10), lambda i: (0, 0, 0)),
                pl.BlockSpec((1, 10), lambda i: (0, 0)),
            ],
            out_specs=pl.BlockSpec((bt, 10), lambda i: (i, 0)),
        ),
        compiler_params=pltpu.CompilerParams(
            dimension_semantics=("parallel",),
            vmem_limit_bytes=56 * 1024 * 1024,
        ),
    )(xpt, m1e, m1o, m2e, m2o, b1row, b2row, wf2, bfc)
